# Initial kernel scaffold; baseline (speedup 1.0000x reference)
import jax
import jax.numpy as jnp
from jax.experimental import pallas as pl


def kernel(x, edge_index, edge_attr, batch, lin1_w, lin1_b, eps1, W1a, b1a, W1b, b1b, lin2_w, lin2_b, eps2, W2a, b2a, W2b, b2b, Wf1, bf1, Wf2, bf2):
    raise NotImplementedError("write your pallas kernel here")


# speedup vs baseline: 1.8142x; 1.8142x over previous
import functools

import jax
import jax.numpy as jnp
from jax import lax
from jax.experimental import pallas as pl
from jax.experimental.pallas import tpu as pltpu
from jax.experimental.pallas import tpu_sc as plsc

_N = 10000
_E = 320000
_D = 128
_DE = 16
_H = 128
_NG = 64

_NC = 2
_NS = 16
_NW = _NC * _NS
_EW = _E // _NW
_C = 80
_NCHUNK = _EW // _C
_NP = 10240
_RT = _NP // _NS
_ZR = 128


_BE = 2000


def _edge_feat_body(ea_ref, w1_ref, b1_ref, w2_ref, b2_ref, e1_ref, e2_ref):
    ea = ea_ref[...]
    e1_ref[...] = jnp.dot(ea, w1_ref[...], preferred_element_type=jnp.float32) + b1_ref[...]
    e2_ref[...] = jnp.dot(ea, w2_ref[...], preferred_element_type=jnp.float32) + b2_ref[...]


def _edge_feats(edge_attr, w1, b1, w2, b2):
    grid = (_E // _BE,)
    return pl.pallas_call(
        _edge_feat_body,
        grid=grid,
        in_specs=[
            pl.BlockSpec((_BE, _DE), lambda i: (i, 0)),
            pl.BlockSpec((_DE, _D), lambda i: (0, 0)),
            pl.BlockSpec((1, _D), lambda i: (0, 0)),
            pl.BlockSpec((_DE, _H), lambda i: (0, 0)),
            pl.BlockSpec((1, _H), lambda i: (0, 0)),
        ],
        out_specs=[
            pl.BlockSpec((_BE, _D), lambda i: (i, 0)),
            pl.BlockSpec((_BE, _H), lambda i: (i, 0)),
        ],
        out_shape=[
            jax.ShapeDtypeStruct((_E, _D), jnp.float32),
            jax.ShapeDtypeStruct((_E, _H), jnp.float32),
        ],
    )(edge_attr, w1, b1.reshape(1, _D), w2, b2.reshape(1, _H))


def _sc_gather_relu(x, e, src):
    mesh = plsc.VectorSubcoreMesh(core_axis_name="c", subcore_axis_name="s")

    @functools.partial(
        pl.kernel,
        out_type=jax.ShapeDtypeStruct((_E, _D), jnp.float32),
        mesh=mesh,
        scratch_types=[
            pltpu.VMEM((_C,), jnp.int32),
            pltpu.VMEM((_C, _D), jnp.float32),
            pltpu.VMEM((_C, _D), jnp.float32),
            pltpu.SemaphoreType.DMA,
            pltpu.SemaphoreType.DMA,
        ],
    )
    def k(x_hbm, e_hbm, src_hbm, out_hbm, src_v, xr_v, e_v, sem_a, sem_b):
        c = lax.axis_index("c")
        s = lax.axis_index("s")
        w = c * _NS + s

        def _chunk(i, carry):
            base = w * _EW + i * _C
            pltpu.sync_copy(src_hbm.at[pl.ds(base, _C)], src_v)
            cp_e = pltpu.async_copy(e_hbm.at[pl.ds(base, _C)], e_v, sem_a)
            cp_x = pltpu.async_copy(x_hbm.at[src_v], xr_v, sem_b)
            cp_e.wait()
            cp_x.wait()

            def _row(r, rc):
                for j in range(_D // 16):
                    v = xr_v[r, pl.ds(j * 16, 16)] + e_v[r, pl.ds(j * 16, 16)]
                    e_v[r, pl.ds(j * 16, 16)] = jnp.maximum(v, 0.0)
                return rc

            lax.fori_loop(0, _C, _row, 0)
            pltpu.sync_copy(e_v, out_hbm.at[pl.ds(base, _C)])
            return carry

        lax.fori_loop(0, _NCHUNK, _chunk, 0)

    return k(x, e, src)


def _sc_messages(x, e, src, dst):
    mesh = plsc.VectorSubcoreMesh(core_axis_name="c", subcore_axis_name="s")

    @functools.partial(
        pl.kernel,
        out_type=jax.ShapeDtypeStruct((_NC, _NP, _D), jnp.float32),
        mesh=mesh,
        scratch_types=[
            pltpu.VMEM_SHARED((_NP, _D), jnp.float32),
            pltpu.VMEM((_C,), jnp.int32),
            pltpu.VMEM((_C,), jnp.int32),
            pltpu.VMEM((_C, _D), jnp.float32),
            pltpu.VMEM((_C, _D), jnp.float32),
            pltpu.VMEM((_ZR, _D), jnp.float32),
            pltpu.SemaphoreType.DMA,
            pltpu.SemaphoreType.DMA,
        ],
    )
    def k(x_hbm, e_hbm, src_hbm, dst_hbm, out_hbm,
          aggr_sh, src_v, dst_v, xr_v, e_v, zb_v, sem_a, sem_b):
        c = lax.axis_index("c")
        s = lax.axis_index("s")
        w = c * _NS + s

        def _zrow(i, carry):
            for j in range(_D // 16):
                zb_v[i, pl.ds(j * 16, 16)] = jnp.zeros((16,), jnp.float32)
            return carry

        lax.fori_loop(0, _ZR, _zrow, 0)
        for b in range(_RT // _ZR):
            pltpu.sync_copy(zb_v, aggr_sh.at[pl.ds(s * _RT + b * _ZR, _ZR)])
        plsc.subcore_barrier()

        def _chunk(i, carry):
            base = w * _EW + i * _C
            pltpu.sync_copy(src_hbm.at[pl.ds(base, _C)], src_v)
            pltpu.sync_copy(dst_hbm.at[pl.ds(base, _C)], dst_v)
            cp_e = pltpu.async_copy(e_hbm.at[pl.ds(base, _C)], e_v, sem_a)
            cp_x = pltpu.async_copy(x_hbm.at[src_v], xr_v, sem_b)
            cp_e.wait()
            cp_x.wait()

            def _row(r, rc):
                for j in range(_D // 16):
                    v = xr_v[r, pl.ds(j * 16, 16)] + e_v[r, pl.ds(j * 16, 16)]
                    e_v[r, pl.ds(j * 16, 16)] = jnp.maximum(v, 0.0)
                return rc

            lax.fori_loop(0, _C, _row, 0)
            pltpu.sync_copy(e_v, aggr_sh.at[dst_v], add=True)
            return carry

        lax.fori_loop(0, _NCHUNK, _chunk, 0)
        plsc.subcore_barrier()

        pltpu.sync_copy(aggr_sh.at[pl.ds(s * _RT, _RT)],
                        out_hbm.at[c, pl.ds(s * _RT, _RT)])

    return k(x, e, src, dst)


_BN = 1000


def _mlp_body(eps_ref, x_ref, a0_ref, wa_ref, ba_ref, wb_ref, bb_ref, o_ref):
    hin = x_ref[...] * eps_ref[0, 0] + a0_ref[...]
    t = jnp.maximum(jnp.dot(hin, wa_ref[...], preferred_element_type=jnp.float32)
                    + ba_ref[...], 0.0)
    h = jnp.maximum(jnp.dot(t, wb_ref[...], preferred_element_type=jnp.float32)
                    + bb_ref[...], 0.0)
    o_ref[...] = h


def _mlp1(x, a0, eps, wa, ba, wb, bb):
    grid = (_N // _BN,)
    return pl.pallas_call(
        _mlp_body,
        grid=grid,
        in_specs=[
            pl.BlockSpec(memory_space=pltpu.SMEM),
            pl.BlockSpec((_BN, _D), lambda i: (i, 0)),
            pl.BlockSpec((_BN, _D), lambda i: (i, 0)),
            pl.BlockSpec((_D, _H), lambda i: (0, 0)),
            pl.BlockSpec((1, _H), lambda i: (0, 0)),
            pl.BlockSpec((_H, _H), lambda i: (0, 0)),
            pl.BlockSpec((1, _H), lambda i: (0, 0)),
        ],
        out_specs=pl.BlockSpec((_BN, _H), lambda i: (i, 0)),
        out_shape=jax.ShapeDtypeStruct((_N, _H), jnp.float32),
    )((1.0 + eps).reshape(1, 1), x, a0, wa, ba.reshape(1, _H), wb, bb.reshape(1, _H))


def _mlp2_body(eps_ref, bf2_ref, x_ref, a0_ref, a1_ref, wa_ref, ba_ref, wb_ref,
               bb_ref, batch_ref, wf1_ref, bf1_ref, wf2_ref, o_ref, pooled_ref):
    i = pl.program_id(0)
    hin = x_ref[...] * eps_ref[0, 0] + a0_ref[...] + a1_ref[...]
    t = jnp.maximum(jnp.dot(hin, wa_ref[...], preferred_element_type=jnp.float32)
                    + ba_ref[...], 0.0)
    h = jnp.maximum(jnp.dot(t, wb_ref[...], preferred_element_type=jnp.float32)
                    + bb_ref[...], 0.0)
    ids = batch_ref[0]
    seg = lax.broadcasted_iota(jnp.int32, (_NG, _BN), 0)
    oh = (seg == ids).astype(jnp.float32)
    part = jnp.dot(oh, h, preferred_element_type=jnp.float32,
                   precision=lax.Precision.HIGHEST)

    @pl.when(i == 0)
    def _():
        pooled_ref[...] = part

    @pl.when(i > 0)
    def _():
        pooled_ref[...] = pooled_ref[...] + part

    @pl.when(i == (_N // _BN) - 1)
    def _():
        p = pooled_ref[...]
        f = jnp.maximum(jnp.dot(p, wf1_ref[...], preferred_element_type=jnp.float32)
                        + bf1_ref[...], 0.0)
        o_ref[...] = jnp.dot(f, wf2_ref[...], preferred_element_type=jnp.float32) \
            + bf2_ref[0, 0]


def _mlp2_pool_fc(x, a0, a1, eps, wa, ba, wb, bb, batch, wf1, bf1, wf2, bf2):
    grid = (_N // _BN,)
    wf2p = jnp.pad(wf2, ((0, 0), (0, _D - wf2.shape[1])))
    batch3 = batch.reshape(_N // _BN, 1, _BN)
    out = pl.pallas_call(
        _mlp2_body,
        grid=grid,
        in_specs=[
            pl.BlockSpec(memory_space=pltpu.SMEM),
            pl.BlockSpec(memory_space=pltpu.SMEM),
            pl.BlockSpec((_BN, _H), lambda i: (i, 0)),
            pl.BlockSpec((_BN, _H), lambda i: (i, 0)),
            pl.BlockSpec((_BN, _H), lambda i: (i, 0)),
            pl.BlockSpec((_H, _H), lambda i: (0, 0)),
            pl.BlockSpec((1, _H), lambda i: (0, 0)),
            pl.BlockSpec((_H, _H), lambda i: (0, 0)),
            pl.BlockSpec((1, _H), lambda i: (0, 0)),
            pl.BlockSpec((1, 1, _BN), lambda i: (i, 0, 0)),
            pl.BlockSpec((_H, 2 * _H), lambda i: (0, 0)),
            pl.BlockSpec((1, 2 * _H), lambda i: (0, 0)),
            pl.BlockSpec((2 * _H, _D), lambda i: (0, 0)),
        ],
        out_specs=pl.BlockSpec((_NG, _D), lambda i: (0, 0)),
        out_shape=jax.ShapeDtypeStruct((_NG, _D), jnp.float32),
        scratch_shapes=[pltpu.VMEM((_NG, _H), jnp.float32)],
    )((1.0 + eps).reshape(1, 1), bf2.reshape(1, 1), x, a0, a1, wa,
      ba.reshape(1, _H), wb, bb.reshape(1, _H), batch3, wf1,
      bf1.reshape(1, 2 * _H), wf2p)
    return out[:, :1]


def kernel(x, edge_index, edge_attr, batch,
           lin1_w, lin1_b, eps1, W1a, b1a, W1b, b1b,
           lin2_w, lin2_b, eps2, W2a, b2a, W2b, b2b,
           Wf1, bf1, Wf2, bf2):
    src = edge_index[0]
    dst = edge_index[1]
    e1, e2 = _edge_feats(edge_attr, lin1_w, lin1_b, lin2_w, lin2_b)

    m1 = _sc_gather_relu(x, e1, src)
    aggr1 = jnp.zeros((_N, _D), jnp.float32).at[dst].add(m1)
    h1 = _mlp1(x, aggr1, eps1, W1a, b1a, W1b, b1b)

    aggr2 = _sc_messages(h1, e2, src, dst)
    return _mlp2_pool_fc(h1, aggr2[0, :_N], aggr2[1, :_N], eps2, W2a, b2a, W2b, b2b,
                         batch, Wf1, bf1, Wf2, bf2)


# speedup vs baseline: 2.0251x; 1.1163x over previous
import functools

import jax
import jax.numpy as jnp
from jax import lax
from jax.experimental import pallas as pl
from jax.experimental.pallas import tpu as pltpu
from jax.experimental.pallas import tpu_sc as plsc

_N = 10000
_E = 320000
_D = 128
_DE = 16
_H = 128
_NG = 64

_NC = 2
_NS = 16
_NW = _NC * _NS
_EW = _E // _NW
_C = 80
_NCHUNK = _EW // _C
_NP = 10240
_RT = _NP // _NS
_ZR = 32


_BE = 2000


def _edge_feat_body(ea_ref, w1_ref, b1_ref, w2_ref, b2_ref, e1_ref, e2_ref):
    ea = ea_ref[...]
    e1_ref[...] = jnp.dot(ea, w1_ref[...], preferred_element_type=jnp.float32) + b1_ref[...]
    e2_ref[...] = jnp.dot(ea, w2_ref[...], preferred_element_type=jnp.float32) + b2_ref[...]


def _edge_feats(edge_attr, w1, b1, w2, b2):
    grid = (_E // _BE,)
    return pl.pallas_call(
        _edge_feat_body,
        grid=grid,
        in_specs=[
            pl.BlockSpec((_BE, _DE), lambda i: (i, 0)),
            pl.BlockSpec((_DE, _D), lambda i: (0, 0)),
            pl.BlockSpec((1, _D), lambda i: (0, 0)),
            pl.BlockSpec((_DE, _H), lambda i: (0, 0)),
            pl.BlockSpec((1, _H), lambda i: (0, 0)),
        ],
        out_specs=[
            pl.BlockSpec((_BE, _D), lambda i: (i, 0)),
            pl.BlockSpec((_BE, _H), lambda i: (i, 0)),
        ],
        out_shape=[
            jax.ShapeDtypeStruct((_E, _D), jnp.float32),
            jax.ShapeDtypeStruct((_E, _H), jnp.float32),
        ],
    )(edge_attr, w1, b1.reshape(1, _D), w2, b2.reshape(1, _H))


def _sc_gather_relu(x, e, src):
    mesh = plsc.VectorSubcoreMesh(core_axis_name="c", subcore_axis_name="s")

    @functools.partial(
        pl.kernel,
        out_type=jax.ShapeDtypeStruct((_E, _D), jnp.float32),
        mesh=mesh,
        scratch_types=[
            [pltpu.VMEM((_C,), jnp.int32)] * 2,
            [pltpu.VMEM((_C, _D), jnp.float32)] * 2,
            [pltpu.VMEM((_C, _D), jnp.float32)] * 2,
            [pltpu.SemaphoreType.DMA] * 2,
            [pltpu.SemaphoreType.DMA] * 2,
        ],
    )
    def k(x_hbm, e_hbm, src_hbm, out_hbm, src_v, xr_v, e_v, sem_e, sem_g):
        c = lax.axis_index("c")
        s = lax.axis_index("s")
        w = c * _NS + s

        def _issue(i, b):
            base = w * _EW + i * _C
            pltpu.sync_copy(src_hbm.at[pl.ds(base, _C)], src_v[b])
            cp_e = pltpu.async_copy(e_hbm.at[pl.ds(base, _C)], e_v[b], sem_e[b])
            cp_x = pltpu.async_copy(x_hbm.at[src_v[b]], xr_v[b], sem_g[b])
            return cp_e, cp_x

        def _compute_store(i, b):
            pltpu.make_async_copy(e_hbm.at[pl.ds(0, _C)], e_v[b], sem_e[b]).wait()
            pltpu.make_async_copy(x_hbm.at[pl.ds(0, _C)], xr_v[b], sem_g[b]).wait()

            def _row(r, rc):
                for j in range(_D // 16):
                    v = xr_v[b][r, pl.ds(j * 16, 16)] + e_v[b][r, pl.ds(j * 16, 16)]
                    e_v[b][r, pl.ds(j * 16, 16)] = jnp.maximum(v, 0.0)
                return rc

            lax.fori_loop(0, _C, _row, 0)
            base = w * _EW + i * _C
            pltpu.sync_copy(e_v[b], out_hbm.at[pl.ds(base, _C)])

        _issue(0, 0)
        _compute_store(0, 0)
        _issue(1, 0)

        def _pair(kk, carry):
            a = 2 * kk + 1
            _issue(a + 1, 1)
            _compute_store(a, 0)

            @pl.when(kk < (_NCHUNK - 1) // 2 - 1)
            def _():
                _issue(a + 2, 0)

            _compute_store(a + 1, 1)
            return carry

        lax.fori_loop(0, (_NCHUNK - 1) // 2, _pair, 0)

    return k(x, e, src)


def _sc_messages(x, e, src, dst):
    mesh = plsc.VectorSubcoreMesh(core_axis_name="c", subcore_axis_name="s")

    @functools.partial(
        pl.kernel,
        out_type=jax.ShapeDtypeStruct((_NC, _NP, _D), jnp.float32),
        mesh=mesh,
        scratch_types=[
            pltpu.VMEM_SHARED((_NP, _D), jnp.float32),
            [pltpu.VMEM((_C,), jnp.int32)] * 2,
            [pltpu.VMEM((_C,), jnp.int32)] * 2,
            [pltpu.VMEM((_C, _D), jnp.float32)] * 2,
            [pltpu.VMEM((_C, _D), jnp.float32)] * 2,
            pltpu.VMEM((_ZR, _D), jnp.float32),
            [pltpu.SemaphoreType.DMA] * 2,
            [pltpu.SemaphoreType.DMA] * 2,
        ],
    )
    def k(x_hbm, e_hbm, src_hbm, dst_hbm, out_hbm,
          aggr_sh, src_v, dst_v, xr_v, e_v, zb_v, sem_e, sem_g):
        c = lax.axis_index("c")
        s = lax.axis_index("s")
        w = c * _NS + s

        def _zrow(i, carry):
            for j in range(_D // 16):
                zb_v[i, pl.ds(j * 16, 16)] = jnp.zeros((16,), jnp.float32)
            return carry

        lax.fori_loop(0, _ZR, _zrow, 0)
        for b in range(_RT // _ZR):
            pltpu.sync_copy(zb_v, aggr_sh.at[pl.ds(s * _RT + b * _ZR, _ZR)])
        plsc.subcore_barrier()

        def _issue(i, b):
            base = w * _EW + i * _C
            pltpu.sync_copy(src_hbm.at[pl.ds(base, _C)], src_v[b])
            pltpu.sync_copy(dst_hbm.at[pl.ds(base, _C)], dst_v[b])
            pltpu.async_copy(e_hbm.at[pl.ds(base, _C)], e_v[b], sem_e[b])
            pltpu.async_copy(x_hbm.at[src_v[b]], xr_v[b], sem_g[b])

        def _compute_scatter(i, b):
            pltpu.make_async_copy(e_hbm.at[pl.ds(0, _C)], e_v[b], sem_e[b]).wait()
            pltpu.make_async_copy(x_hbm.at[pl.ds(0, _C)], xr_v[b], sem_g[b]).wait()

            def _row(r, rc):
                for j in range(_D // 16):
                    v = xr_v[b][r, pl.ds(j * 16, 16)] + e_v[b][r, pl.ds(j * 16, 16)]
                    e_v[b][r, pl.ds(j * 16, 16)] = jnp.maximum(v, 0.0)
                return rc

            lax.fori_loop(0, _C, _row, 0)
            pltpu.sync_copy(e_v[b], aggr_sh.at[dst_v[b]], add=True)

        _issue(0, 0)
        _compute_scatter(0, 0)
        _issue(1, 0)

        def _pair(kk, carry):
            a = 2 * kk + 1
            _issue(a + 1, 1)
            _compute_scatter(a, 0)

            @pl.when(kk < (_NCHUNK - 1) // 2 - 1)
            def _():
                _issue(a + 2, 0)

            _compute_scatter(a + 1, 1)
            return carry

        lax.fori_loop(0, (_NCHUNK - 1) // 2, _pair, 0)
        plsc.subcore_barrier()

        pltpu.sync_copy(aggr_sh.at[pl.ds(s * _RT, _RT)],
                        out_hbm.at[c, pl.ds(s * _RT, _RT)])

    return k(x, e, src, dst)


_BN = 1000


def _mlp_body(eps_ref, x_ref, a0_ref, wa_ref, ba_ref, wb_ref, bb_ref, o_ref):
    hin = x_ref[...] * eps_ref[0, 0] + a0_ref[...]
    t = jnp.maximum(jnp.dot(hin, wa_ref[...], preferred_element_type=jnp.float32)
                    + ba_ref[...], 0.0)
    h = jnp.maximum(jnp.dot(t, wb_ref[...], preferred_element_type=jnp.float32)
                    + bb_ref[...], 0.0)
    o_ref[...] = h


def _mlp1(x, a0, eps, wa, ba, wb, bb):
    grid = (_N // _BN,)
    return pl.pallas_call(
        _mlp_body,
        grid=grid,
        in_specs=[
            pl.BlockSpec(memory_space=pltpu.SMEM),
            pl.BlockSpec((_BN, _D), lambda i: (i, 0)),
            pl.BlockSpec((_BN, _D), lambda i: (i, 0)),
            pl.BlockSpec((_D, _H), lambda i: (0, 0)),
            pl.BlockSpec((1, _H), lambda i: (0, 0)),
            pl.BlockSpec((_H, _H), lambda i: (0, 0)),
            pl.BlockSpec((1, _H), lambda i: (0, 0)),
        ],
        out_specs=pl.BlockSpec((_BN, _H), lambda i: (i, 0)),
        out_shape=jax.ShapeDtypeStruct((_N, _H), jnp.float32),
    )((1.0 + eps).reshape(1, 1), x, a0, wa, ba.reshape(1, _H), wb, bb.reshape(1, _H))


def _mlp2_body(eps_ref, bf2_ref, x_ref, a0_ref, a1_ref, wa_ref, ba_ref, wb_ref,
               bb_ref, batch_ref, wf1_ref, bf1_ref, wf2_ref, o_ref, pooled_ref):
    i = pl.program_id(0)
    hin = x_ref[...] * eps_ref[0, 0] + a0_ref[...] + a1_ref[...]
    t = jnp.maximum(jnp.dot(hin, wa_ref[...], preferred_element_type=jnp.float32)
                    + ba_ref[...], 0.0)
    h = jnp.maximum(jnp.dot(t, wb_ref[...], preferred_element_type=jnp.float32)
                    + bb_ref[...], 0.0)
    ids = batch_ref[0]
    seg = lax.broadcasted_iota(jnp.int32, (_NG, _BN), 0)
    oh = (seg == ids).astype(jnp.float32)
    part = jnp.dot(oh, h, preferred_element_type=jnp.float32,
                   precision=lax.Precision.HIGHEST)

    @pl.when(i == 0)
    def _():
        pooled_ref[...] = part

    @pl.when(i > 0)
    def _():
        pooled_ref[...] = pooled_ref[...] + part

    @pl.when(i == (_N // _BN) - 1)
    def _():
        p = pooled_ref[...]
        f = jnp.maximum(jnp.dot(p, wf1_ref[...], preferred_element_type=jnp.float32)
                        + bf1_ref[...], 0.0)
        o_ref[...] = jnp.dot(f, wf2_ref[...], preferred_element_type=jnp.float32) \
            + bf2_ref[0, 0]


def _mlp2_pool_fc(x, a0, a1, eps, wa, ba, wb, bb, batch, wf1, bf1, wf2, bf2):
    grid = (_N // _BN,)
    wf2p = jnp.pad(wf2, ((0, 0), (0, _D - wf2.shape[1])))
    batch3 = batch.reshape(_N // _BN, 1, _BN)
    out = pl.pallas_call(
        _mlp2_body,
        grid=grid,
        in_specs=[
            pl.BlockSpec(memory_space=pltpu.SMEM),
            pl.BlockSpec(memory_space=pltpu.SMEM),
            pl.BlockSpec((_BN, _H), lambda i: (i, 0)),
            pl.BlockSpec((_BN, _H), lambda i: (i, 0)),
            pl.BlockSpec((_BN, _H), lambda i: (i, 0)),
            pl.BlockSpec((_H, _H), lambda i: (0, 0)),
            pl.BlockSpec((1, _H), lambda i: (0, 0)),
            pl.BlockSpec((_H, _H), lambda i: (0, 0)),
            pl.BlockSpec((1, _H), lambda i: (0, 0)),
            pl.BlockSpec((1, 1, _BN), lambda i: (i, 0, 0)),
            pl.BlockSpec((_H, 2 * _H), lambda i: (0, 0)),
            pl.BlockSpec((1, 2 * _H), lambda i: (0, 0)),
            pl.BlockSpec((2 * _H, _D), lambda i: (0, 0)),
        ],
        out_specs=pl.BlockSpec((_NG, _D), lambda i: (0, 0)),
        out_shape=jax.ShapeDtypeStruct((_NG, _D), jnp.float32),
        scratch_shapes=[pltpu.VMEM((_NG, _H), jnp.float32)],
    )((1.0 + eps).reshape(1, 1), bf2.reshape(1, 1), x, a0, a1, wa,
      ba.reshape(1, _H), wb, bb.reshape(1, _H), batch3, wf1,
      bf1.reshape(1, 2 * _H), wf2p)
    return out[:, :1]


def kernel(x, edge_index, edge_attr, batch,
           lin1_w, lin1_b, eps1, W1a, b1a, W1b, b1b,
           lin2_w, lin2_b, eps2, W2a, b2a, W2b, b2b,
           Wf1, bf1, Wf2, bf2):
    src = edge_index[0]
    dst = edge_index[1]
    e1, e2 = _edge_feats(edge_attr, lin1_w, lin1_b, lin2_w, lin2_b)

    m1 = _sc_gather_relu(x, e1, src)
    aggr1 = jnp.zeros((_N, _D), jnp.float32).at[dst].add(m1)
    h1 = _mlp1(x, aggr1, eps1, W1a, b1a, W1b, b1b)

    aggr2 = _sc_messages(h1, e2, src, dst)
    return _mlp2_pool_fc(h1, aggr2[0, :_N], aggr2[1, :_N], eps2, W2a, b2a, W2b, b2b,
                         batch, Wf1, bf1, Wf2, bf2)


# speedup vs baseline: 2.0715x; 1.0229x over previous
import functools

import jax
import jax.numpy as jnp
from jax import lax
from jax.experimental import pallas as pl
from jax.experimental.pallas import tpu as pltpu
from jax.experimental.pallas import tpu_sc as plsc

_N = 10000
_E = 320000
_D = 128
_DE = 16
_H = 128
_NG = 64

_NC = 2
_NS = 16
_NW = _NC * _NS
_EW = _E // _NW
_C = 80
_NCHUNK = _EW // _C
_NP = 10240
_RT = _NP // _NS
_ZR = 32


_BE = 2000


def _edge_feat_body(ea_ref, w_ref, b_ref, e_ref):
    ea = ea_ref[...]
    e_ref[...] = jnp.dot(ea, w_ref[...], preferred_element_type=jnp.float32) + b_ref[...]


def _edge_feats(edge_attr, w, b):
    grid = (_E // _BE,)
    return pl.pallas_call(
        _edge_feat_body,
        grid=grid,
        in_specs=[
            pl.BlockSpec((_BE, _DE), lambda i: (i, 0)),
            pl.BlockSpec((_DE, _D), lambda i: (0, 0)),
            pl.BlockSpec((1, _D), lambda i: (0, 0)),
        ],
        out_specs=pl.BlockSpec((_BE, _D), lambda i: (i, 0)),
        out_shape=jax.ShapeDtypeStruct((_E, _D), jnp.float32),
    )(edge_attr, w, b.reshape(1, _D))


def _sc_gather_relu(x, e, src):
    mesh = plsc.VectorSubcoreMesh(core_axis_name="c", subcore_axis_name="s")

    @functools.partial(
        pl.kernel,
        out_type=jax.ShapeDtypeStruct((_E, _D), jnp.float32),
        mesh=mesh,
        scratch_types=[
            [pltpu.VMEM((_C,), jnp.int32)] * 2,
            [pltpu.VMEM((_C, _D), jnp.float32)] * 2,
            [pltpu.VMEM((_C, _D), jnp.float32)] * 2,
            [pltpu.SemaphoreType.DMA] * 2,
            [pltpu.SemaphoreType.DMA] * 2,
        ],
    )
    def k(x_hbm, e_hbm, src_hbm, out_hbm, src_v, xr_v, e_v, sem_e, sem_g):
        c = lax.axis_index("c")
        s = lax.axis_index("s")
        w = c * _NS + s

        def _issue(i, b):
            base = w * _EW + i * _C
            pltpu.sync_copy(src_hbm.at[pl.ds(base, _C)], src_v[b])
            cp_e = pltpu.async_copy(e_hbm.at[pl.ds(base, _C)], e_v[b], sem_e[b])
            cp_x = pltpu.async_copy(x_hbm.at[src_v[b]], xr_v[b], sem_g[b])
            return cp_e, cp_x

        def _compute_store(i, b):
            pltpu.make_async_copy(e_hbm.at[pl.ds(0, _C)], e_v[b], sem_e[b]).wait()
            pltpu.make_async_copy(x_hbm.at[pl.ds(0, _C)], xr_v[b], sem_g[b]).wait()

            def _row(r, rc):
                for j in range(_D // 16):
                    v = xr_v[b][r, pl.ds(j * 16, 16)] + e_v[b][r, pl.ds(j * 16, 16)]
                    e_v[b][r, pl.ds(j * 16, 16)] = jnp.maximum(v, 0.0)
                return rc

            lax.fori_loop(0, _C, _row, 0)
            base = w * _EW + i * _C
            pltpu.sync_copy(e_v[b], out_hbm.at[pl.ds(base, _C)])

        _issue(0, 0)
        _compute_store(0, 0)
        _issue(1, 0)

        def _pair(kk, carry):
            a = 2 * kk + 1
            _issue(a + 1, 1)
            _compute_store(a, 0)

            @pl.when(kk < (_NCHUNK - 1) // 2 - 1)
            def _():
                _issue(a + 2, 0)

            _compute_store(a + 1, 1)
            return carry

        lax.fori_loop(0, (_NCHUNK - 1) // 2, _pair, 0)

    return k(x, e, src)


def _sc_messages(x, e, src, dst):
    mesh = plsc.VectorSubcoreMesh(core_axis_name="c", subcore_axis_name="s")

    @functools.partial(
        pl.kernel,
        out_type=jax.ShapeDtypeStruct((_NC, _NP, _D), jnp.float32),
        mesh=mesh,
        scratch_types=[
            pltpu.VMEM_SHARED((_NP, _D), jnp.float32),
            [pltpu.VMEM((_C,), jnp.int32)] * 2,
            [pltpu.VMEM((_C,), jnp.int32)] * 2,
            [pltpu.VMEM((_C, _D), jnp.float32)] * 2,
            [pltpu.VMEM((_C, _D), jnp.float32)] * 2,
            pltpu.VMEM((_ZR, _D), jnp.float32),
            [pltpu.SemaphoreType.DMA] * 2,
            [pltpu.SemaphoreType.DMA] * 2,
        ],
    )
    def k(x_hbm, e_hbm, src_hbm, dst_hbm, out_hbm,
          aggr_sh, src_v, dst_v, xr_v, e_v, zb_v, sem_e, sem_g):
        c = lax.axis_index("c")
        s = lax.axis_index("s")
        w = c * _NS + s

        def _zrow(i, carry):
            for j in range(_D // 16):
                zb_v[i, pl.ds(j * 16, 16)] = jnp.zeros((16,), jnp.float32)
            return carry

        lax.fori_loop(0, _ZR, _zrow, 0)
        for b in range(_RT // _ZR):
            pltpu.sync_copy(zb_v, aggr_sh.at[pl.ds(s * _RT + b * _ZR, _ZR)])
        plsc.subcore_barrier()

        def _issue(i, b):
            base = w * _EW + i * _C
            pltpu.sync_copy(src_hbm.at[pl.ds(base, _C)], src_v[b])
            pltpu.sync_copy(dst_hbm.at[pl.ds(base, _C)], dst_v[b])
            pltpu.async_copy(e_hbm.at[pl.ds(base, _C)], e_v[b], sem_e[b])
            pltpu.async_copy(x_hbm.at[src_v[b]], xr_v[b], sem_g[b])

        def _compute_scatter(i, b):
            pltpu.make_async_copy(e_hbm.at[pl.ds(0, _C)], e_v[b], sem_e[b]).wait()
            pltpu.make_async_copy(x_hbm.at[pl.ds(0, _C)], xr_v[b], sem_g[b]).wait()

            def _row(r, rc):
                for j in range(_D // 16):
                    v = xr_v[b][r, pl.ds(j * 16, 16)] + e_v[b][r, pl.ds(j * 16, 16)]
                    e_v[b][r, pl.ds(j * 16, 16)] = jnp.maximum(v, 0.0)
                return rc

            lax.fori_loop(0, _C, _row, 0)
            pltpu.sync_copy(e_v[b], aggr_sh.at[dst_v[b]], add=True)

        _issue(0, 0)
        _compute_scatter(0, 0)
        _issue(1, 0)

        def _pair(kk, carry):
            a = 2 * kk + 1
            _issue(a + 1, 1)
            _compute_scatter(a, 0)

            @pl.when(kk < (_NCHUNK - 1) // 2 - 1)
            def _():
                _issue(a + 2, 0)

            _compute_scatter(a + 1, 1)
            return carry

        lax.fori_loop(0, (_NCHUNK - 1) // 2, _pair, 0)
        plsc.subcore_barrier()

        pltpu.sync_copy(aggr_sh.at[pl.ds(s * _RT, _RT)],
                        out_hbm.at[c, pl.ds(s * _RT, _RT)])

    return k(x, e, src, dst)


_BN = 1000


def _mlp_body(eps_ref, x_ref, a0_ref, wa_ref, ba_ref, wb_ref, bb_ref, o_ref):
    hin = x_ref[...] * eps_ref[0, 0] + a0_ref[...]
    t = jnp.maximum(jnp.dot(hin, wa_ref[...], preferred_element_type=jnp.float32)
                    + ba_ref[...], 0.0)
    h = jnp.maximum(jnp.dot(t, wb_ref[...], preferred_element_type=jnp.float32)
                    + bb_ref[...], 0.0)
    o_ref[...] = h


def _mlp1(x, a0, eps, wa, ba, wb, bb):
    grid = (_N // _BN,)
    return pl.pallas_call(
        _mlp_body,
        grid=grid,
        in_specs=[
            pl.BlockSpec(memory_space=pltpu.SMEM),
            pl.BlockSpec((_BN, _D), lambda i: (i, 0)),
            pl.BlockSpec((_BN, _D), lambda i: (i, 0)),
            pl.BlockSpec((_D, _H), lambda i: (0, 0)),
            pl.BlockSpec((1, _H), lambda i: (0, 0)),
            pl.BlockSpec((_H, _H), lambda i: (0, 0)),
            pl.BlockSpec((1, _H), lambda i: (0, 0)),
        ],
        out_specs=pl.BlockSpec((_BN, _H), lambda i: (i, 0)),
        out_shape=jax.ShapeDtypeStruct((_N, _H), jnp.float32),
    )((1.0 + eps).reshape(1, 1), x, a0, wa, ba.reshape(1, _H), wb, bb.reshape(1, _H))


def _mlp2_body(eps_ref, bf2_ref, x_ref, a0_ref, a1_ref, wa_ref, ba_ref, wb_ref,
               bb_ref, batch_ref, wf1_ref, bf1_ref, wf2_ref, o_ref, pooled_ref):
    i = pl.program_id(0)
    hin = x_ref[...] * eps_ref[0, 0] + a0_ref[...] + a1_ref[...]
    t = jnp.maximum(jnp.dot(hin, wa_ref[...], preferred_element_type=jnp.float32)
                    + ba_ref[...], 0.0)
    h = jnp.maximum(jnp.dot(t, wb_ref[...], preferred_element_type=jnp.float32)
                    + bb_ref[...], 0.0)
    ids = batch_ref[0]
    seg = lax.broadcasted_iota(jnp.int32, (_NG, _BN), 0)
    oh = (seg == ids).astype(jnp.float32)
    part = jnp.dot(oh, h, preferred_element_type=jnp.float32,
                   precision=lax.Precision.HIGHEST)

    @pl.when(i == 0)
    def _():
        pooled_ref[...] = part

    @pl.when(i > 0)
    def _():
        pooled_ref[...] = pooled_ref[...] + part

    @pl.when(i == (_N // _BN) - 1)
    def _():
        p = pooled_ref[...]
        f = jnp.maximum(jnp.dot(p, wf1_ref[...], preferred_element_type=jnp.float32)
                        + bf1_ref[...], 0.0)
        o_ref[...] = jnp.dot(f, wf2_ref[...], preferred_element_type=jnp.float32) \
            + bf2_ref[0, 0]


def _mlp2_pool_fc(x, a0, a1, eps, wa, ba, wb, bb, batch, wf1, bf1, wf2, bf2):
    grid = (_N // _BN,)
    wf2p = jnp.pad(wf2, ((0, 0), (0, _D - wf2.shape[1])))
    batch3 = batch.reshape(_N // _BN, 1, _BN)
    out = pl.pallas_call(
        _mlp2_body,
        grid=grid,
        in_specs=[
            pl.BlockSpec(memory_space=pltpu.SMEM),
            pl.BlockSpec(memory_space=pltpu.SMEM),
            pl.BlockSpec((_BN, _H), lambda i: (i, 0)),
            pl.BlockSpec((_BN, _H), lambda i: (i, 0)),
            pl.BlockSpec((_BN, _H), lambda i: (i, 0)),
            pl.BlockSpec((_H, _H), lambda i: (0, 0)),
            pl.BlockSpec((1, _H), lambda i: (0, 0)),
            pl.BlockSpec((_H, _H), lambda i: (0, 0)),
            pl.BlockSpec((1, _H), lambda i: (0, 0)),
            pl.BlockSpec((1, 1, _BN), lambda i: (i, 0, 0)),
            pl.BlockSpec((_H, 2 * _H), lambda i: (0, 0)),
            pl.BlockSpec((1, 2 * _H), lambda i: (0, 0)),
            pl.BlockSpec((2 * _H, _D), lambda i: (0, 0)),
        ],
        out_specs=pl.BlockSpec((_NG, _D), lambda i: (0, 0)),
        out_shape=jax.ShapeDtypeStruct((_NG, _D), jnp.float32),
        scratch_shapes=[pltpu.VMEM((_NG, _H), jnp.float32)],
    )((1.0 + eps).reshape(1, 1), bf2.reshape(1, 1), x, a0, a1, wa,
      ba.reshape(1, _H), wb, bb.reshape(1, _H), batch3, wf1,
      bf1.reshape(1, 2 * _H), wf2p)
    return out[:, :1]


def kernel(x, edge_index, edge_attr, batch,
           lin1_w, lin1_b, eps1, W1a, b1a, W1b, b1b,
           lin2_w, lin2_b, eps2, W2a, b2a, W2b, b2b,
           Wf1, bf1, Wf2, bf2):
    src = edge_index[0]
    dst = edge_index[1]
    e1 = _edge_feats(edge_attr, lin1_w, lin1_b)
    e2 = _edge_feats(edge_attr, lin2_w, lin2_b)

    m1 = _sc_gather_relu(x, e1, src)
    aggr1 = jnp.zeros((_N, _D), jnp.float32).at[dst].add(m1)
    h1 = _mlp1(x, aggr1, eps1, W1a, b1a, W1b, b1b)

    aggr2 = _sc_messages(h1, e2, src, dst)
    return _mlp2_pool_fc(h1, aggr2[0, :_N], aggr2[1, :_N], eps2, W2a, b2a, W2b, b2b,
                         batch, Wf1, bf1, Wf2, bf2)
